# baseline (speedup 1.0000x reference)
import functools

import jax
import jax.numpy as jnp
from jax import lax
from jax.experimental import pallas as pl
from jax.experimental.pallas import tpu as pltpu
from jax.experimental.pallas import tpu_sc as plsc

_NC = 2
_NS = 16
_NW = _NC * _NS
_CHUNK = 128
_TE = 2048


def _msg_body(eat_ref, xst_ref, w1t_ref, b1_ref, w2pt_ref, b2p_ref, o_ref,
              *, d, e, te):
    h1 = jnp.dot(w1t_ref[...], eat_ref[...],
                 preferred_element_type=jnp.float32) + b1_ref[...]
    hid = 0.5 * h1 * (1.0 + lax.erf(h1 * 0.7071067811865476))
    q = jnp.dot(w2pt_ref[...], hid.astype(jnp.bfloat16),
                preferred_element_type=jnp.float32) + b2p_ref[...]
    q3 = q.reshape(d, d, te)
    xs3 = xst_ref[...].reshape(d, 1, te)
    msg = jnp.sum(q3 * xs3, axis=0)
    col = pl.program_id(0) * te + lax.broadcasted_iota(jnp.int32, (d, te), 1)
    o_ref[...] = jnp.where(col < e, msg, 0.0)


def _gru_body(xt_ref, aggt_ref, wiht_ref, bih_ref, whht_ref, bhh_ref,
              o_ref, *, d):
    xt = xt_ref[...]
    aggt = aggt_ref[0]
    gi = jnp.dot(wiht_ref[...], aggt,
                 preferred_element_type=jnp.float32) + bih_ref[...]
    gh = jnp.dot(whht_ref[...], xt,
                 preferred_element_type=jnp.float32) + bhh_ref[...]
    r = jax.nn.sigmoid(gi[:d] + gh[:d])
    z = jax.nn.sigmoid(gi[d:2 * d] + gh[d:2 * d])
    hn = gh[2 * d:]
    n = jnp.tanh(gi[2 * d:] + r * hn)
    o_ref[...] = (1.0 - z) * n + z * xt


def _make_gather(n, d, e_pad):
    per_w = e_pad // _NW
    n_ch = per_w // _CHUNK
    mesh = plsc.VectorSubcoreMesh(core_axis_name="c", subcore_axis_name="s")

    @functools.partial(
        pl.kernel, mesh=mesh,
        out_type=jax.ShapeDtypeStruct((e_pad, d), jnp.float32),
        compiler_params=pltpu.CompilerParams(use_tc_tiling_on_sc=False),
        scratch_types=[
            pltpu.VMEM((n_ch, _CHUNK), jnp.int32),
            pltpu.VMEM((per_w, d), jnp.float32),
            pltpu.SemaphoreType.DMA,
        ],
    )
    def gather_k(x_hbm, idx_hbm, out_hbm, idx_v, rows_v, sem):
        w = lax.axis_index("c") * _NS + lax.axis_index("s")
        pltpu.sync_copy(idx_hbm.at[w], idx_v)
        cps = [
            pltpu.async_copy(x_hbm.at[idx_v.at[ch]],
                             rows_v.at[pl.ds(ch * _CHUNK, _CHUNK)], sem)
            for ch in range(n_ch)
        ]
        for cp in cps:
            cp.wait()
        pltpu.sync_copy(rows_v, out_hbm.at[pl.ds(w * per_w, per_w)])

    return gather_k


def _make_scatter(n_acc, d, e_pad):
    per_w = e_pad // _NS
    n_st = 5
    stage = per_w // n_st
    n_ch = stage // _CHUNK
    rows_per_tile = n_acc // _NS
    mesh = plsc.VectorSubcoreMesh(core_axis_name="c", subcore_axis_name="s")

    @functools.partial(
        pl.kernel, mesh=mesh,
        out_type=jax.ShapeDtypeStruct((_NC, n_acc, d), jnp.float32),
        compiler_params=pltpu.CompilerParams(use_tc_tiling_on_sc=False),
        scratch_types=[
            pltpu.VMEM((stage, d), jnp.float32),
            pltpu.VMEM((n_st * n_ch, _CHUNK), jnp.int32),
            pltpu.VMEM_SHARED((n_acc, d), jnp.float32),
            pltpu.SemaphoreType.DMA,
        ],
    )
    def scatter_k(msg_hbm, idx_hbm, zeros_hbm, out_hbm, upd_v, idx_v, agg_sh,
                  sem):
        c = lax.axis_index("c")
        s = lax.axis_index("s")
        pltpu.sync_copy(zeros_hbm,
                        agg_sh.at[pl.ds(s * rows_per_tile, rows_per_tile)])
        pltpu.sync_copy(idx_hbm.at[c, s], idx_v)
        plsc.subcore_barrier()
        for st in range(n_st):
            pltpu.sync_copy(
                msg_hbm.at[pl.ds(s * per_w + st * stage, stage)], upd_v)
            cps = [
                pltpu.async_copy(upd_v.at[pl.ds(ch * _CHUNK, _CHUNK)],
                                 agg_sh.at[idx_v.at[st * n_ch + ch]], sem,
                                 add=True)
                for ch in range(n_ch)
            ]
            for cp in cps:
                cp.wait()
        plsc.subcore_barrier()
        pltpu.sync_copy(
            agg_sh.at[pl.ds(s * rows_per_tile, rows_per_tile)],
            out_hbm.at[c, pl.ds(s * rows_per_tile, rows_per_tile)])

    return scatter_k


def kernel(x, edge_index, edge_attr, W1, b1, W2, b2, W_ih, b_ih, W_hh, b_hh):
    n, d = x.shape
    e = edge_index.shape[1]
    ed = edge_attr.shape[1]
    h = W1.shape[1]

    quant = _NW * _CHUNK
    e_pad = ((e + quant - 1) // quant) * quant
    pad = e_pad - e
    n_ch = e_pad // _NW // _CHUNK

    src = edge_index[0]
    dst = edge_index[1]
    fill = (jnp.arange(pad, dtype=jnp.int32) * 61) % n
    src_p = jnp.concatenate([src, fill]).reshape(_NW, n_ch, _CHUNK)
    dst_p = jnp.concatenate([dst, fill])

    tn = 1024
    split = ((n + _NC * tn - 1) // (_NC * tn)) * tn
    n_acc = ((split + _CHUNK + 15) // 16) * 16
    dump = split + (jnp.arange(e_pad, dtype=jnp.int32) % _CHUNK)
    idx_sc = []
    for c in range(_NC):
        local = dst_p - c * split
        valid = (local >= 0) & (dst_p < min(n, (c + 1) * split))
        idx_sc.append(jnp.where(valid, local, dump))
    idx_sc = jnp.stack(idx_sc).reshape(_NC, _NS, e_pad // _NS // _CHUNK,
                                       _CHUNK)

    ea_t = jnp.pad(edge_attr, ((0, pad), (0, 0))).T
    w2pt = W2.reshape(h, d, d).transpose(2, 1, 0).reshape(d * d, h)
    b2p = b2.reshape(d, d).T.reshape(d * d, 1)

    x_src = _make_gather(n, d, e_pad)(x, src_p)
    xs_t = x_src.T

    grid = e_pad // _TE
    msg_t = pl.pallas_call(
        functools.partial(_msg_body, d=d, e=e, te=_TE),
        grid=(grid,),
        in_specs=[
            pl.BlockSpec((ed, _TE), lambda i: (0, i)),
            pl.BlockSpec((d, _TE), lambda i: (0, i)),
            pl.BlockSpec((h, ed), lambda i: (0, 0)),
            pl.BlockSpec((h, 1), lambda i: (0, 0)),
            pl.BlockSpec((d * d, h), lambda i: (0, 0)),
            pl.BlockSpec((d * d, 1), lambda i: (0, 0)),
        ],
        out_specs=pl.BlockSpec((d, _TE), lambda i: (0, i)),
        out_shape=jax.ShapeDtypeStruct((d, e_pad), jnp.float32),
    )(ea_t, xs_t, W1.T, b1.reshape(h, 1), w2pt.astype(jnp.bfloat16), b2p)
    messages = msg_t.T

    zeros = jnp.zeros((n_acc // _NS, d), jnp.float32)
    agg = _make_scatter(n_acc, d, e_pad)(messages, idx_sc, zeros)

    hb = split // tn
    agg_t = jnp.transpose(agg, (0, 2, 1))
    out_t = pl.pallas_call(
        functools.partial(_gru_body, d=d),
        grid=((n + tn - 1) // tn,),
        in_specs=[
            pl.BlockSpec((d, tn), lambda i: (0, i)),
            pl.BlockSpec((1, d, tn), lambda i: (i // hb, 0, i % hb)),
            pl.BlockSpec((3 * d, d), lambda i: (0, 0)),
            pl.BlockSpec((3 * d, 1), lambda i: (0, 0)),
            pl.BlockSpec((3 * d, d), lambda i: (0, 0)),
            pl.BlockSpec((3 * d, 1), lambda i: (0, 0)),
        ],
        out_specs=pl.BlockSpec((d, tn), lambda i: (0, i)),
        out_shape=jax.ShapeDtypeStruct((d, n), jnp.float32),
    )(x.T, agg_t, W_ih.T, b_ih.reshape(3 * d, 1),
      W_hh.T, b_hh.reshape(3 * d, 1))
    return out_t.T


# speedup vs baseline: 1.0164x; 1.0164x over previous
import functools

import jax
import jax.numpy as jnp
from jax import lax
from jax.experimental import pallas as pl
from jax.experimental.pallas import tpu as pltpu
from jax.experimental.pallas import tpu_sc as plsc

_NC = 2
_NS = 16
_NW = _NC * _NS
_CHUNK = 128
_TE = 2048


def _msg_body(eat_ref, xst_ref, w1t_ref, b1_ref, w2pt_ref, b2p_ref, o_ref,
              *, d, e, te):
    h1 = jnp.dot(w1t_ref[...], eat_ref[...],
                 preferred_element_type=jnp.float32) + b1_ref[...]
    hid = 0.5 * h1 * (1.0 + lax.erf(h1 * 0.7071067811865476))
    q = jnp.dot(w2pt_ref[...], hid.astype(jnp.bfloat16),
                preferred_element_type=jnp.float32) + b2p_ref[...]
    q3 = q.reshape(d, d, te)
    xs3 = xst_ref[...].reshape(d, 1, te)
    msg = jnp.sum(q3 * xs3, axis=0)
    col = pl.program_id(0) * te + lax.broadcasted_iota(jnp.int32, (d, te), 1)
    o_ref[...] = jnp.where(col < e, msg, 0.0)


def _gru_body(xt_ref, aggt_ref, wiht_ref, bih_ref, whht_ref, bhh_ref,
              o_ref, *, d):
    xt = xt_ref[...]
    aggt = aggt_ref[0]
    gi = jnp.dot(wiht_ref[...], aggt,
                 preferred_element_type=jnp.float32) + bih_ref[...]
    gh = jnp.dot(whht_ref[...], xt,
                 preferred_element_type=jnp.float32) + bhh_ref[...]
    r = jax.nn.sigmoid(gi[:d] + gh[:d])
    z = jax.nn.sigmoid(gi[d:2 * d] + gh[d:2 * d])
    hn = gh[2 * d:]
    n = jnp.tanh(gi[2 * d:] + r * hn)
    o_ref[...] = (1.0 - z) * n + z * xt


def _make_gather(n, d, e_pad):
    per_w = e_pad // _NW
    n_ch = per_w // _CHUNK
    mesh = plsc.VectorSubcoreMesh(core_axis_name="c", subcore_axis_name="s")

    @functools.partial(
        pl.kernel, mesh=mesh,
        out_type=jax.ShapeDtypeStruct((e_pad, d), jnp.float32),
        compiler_params=pltpu.CompilerParams(use_tc_tiling_on_sc=False),
        scratch_types=[
            pltpu.VMEM((n_ch, _CHUNK), jnp.int32),
            pltpu.VMEM((per_w, d), jnp.float32),
            pltpu.SemaphoreType.DMA,
            pltpu.SemaphoreType.DMA,
        ],
    )
    def gather_k(x_hbm, idx_hbm, out_hbm, idx_v, rows_v, sem, sem_out):
        w = lax.axis_index("c") * _NS + lax.axis_index("s")
        pltpu.sync_copy(idx_hbm.at[w], idx_v)
        cps = [
            pltpu.async_copy(x_hbm.at[idx_v.at[ch]],
                             rows_v.at[pl.ds(ch * _CHUNK, _CHUNK)], sem)
            for ch in range(n_ch)
        ]
        outs = []
        grp = 5 * _CHUNK
        for g in range(n_ch // 5):
            for ch in range(5 * g, 5 * g + 5):
                cps[ch].wait()
            outs.append(pltpu.async_copy(
                rows_v.at[pl.ds(g * grp, grp)],
                out_hbm.at[pl.ds(w * per_w + g * grp, grp)], sem_out))
        for cp in outs:
            cp.wait()

    return gather_k


def _make_scatter(n_acc, d, e_pad):
    per_w = e_pad // _NS
    n_st = 10
    stage = per_w // n_st
    n_ch = stage // _CHUNK
    rows_per_tile = n_acc // _NS
    mesh = plsc.VectorSubcoreMesh(core_axis_name="c", subcore_axis_name="s")

    @functools.partial(
        pl.kernel, mesh=mesh,
        out_type=jax.ShapeDtypeStruct((_NC, n_acc, d), jnp.float32),
        compiler_params=pltpu.CompilerParams(use_tc_tiling_on_sc=False),
        scratch_types=[
            pltpu.VMEM((2, stage, d), jnp.float32),
            pltpu.VMEM((n_st * n_ch, _CHUNK), jnp.int32),
            pltpu.VMEM_SHARED((n_acc, d), jnp.float32),
            pltpu.SemaphoreType.DMA,
            pltpu.SemaphoreType.DMA,
        ],
    )
    def scatter_k(msg_hbm, idx_hbm, zeros_hbm, out_hbm, upd_v, idx_v, agg_sh,
                  sem, sem_in):
        c = lax.axis_index("c")
        s = lax.axis_index("s")
        pltpu.sync_copy(zeros_hbm,
                        agg_sh.at[pl.ds(s * rows_per_tile, rows_per_tile)])
        pltpu.sync_copy(idx_hbm.at[c, s], idx_v)
        plsc.subcore_barrier()
        loads = [pltpu.async_copy(msg_hbm.at[pl.ds(s * per_w, stage)],
                                  upd_v.at[0], sem_in)]
        for st in range(n_st):
            if st + 1 < n_st:
                loads.append(pltpu.async_copy(
                    msg_hbm.at[pl.ds(s * per_w + (st + 1) * stage, stage)],
                    upd_v.at[(st + 1) % 2], sem_in))
            loads[st].wait()
            cps = [
                pltpu.async_copy(upd_v.at[st % 2, pl.ds(ch * _CHUNK, _CHUNK)],
                                 agg_sh.at[idx_v.at[st * n_ch + ch]], sem,
                                 add=True)
                for ch in range(n_ch)
            ]
            for cp in cps:
                cp.wait()
        plsc.subcore_barrier()
        pltpu.sync_copy(
            agg_sh.at[pl.ds(s * rows_per_tile, rows_per_tile)],
            out_hbm.at[c, pl.ds(s * rows_per_tile, rows_per_tile)])

    return scatter_k


def kernel(x, edge_index, edge_attr, W1, b1, W2, b2, W_ih, b_ih, W_hh, b_hh):
    n, d = x.shape
    e = edge_index.shape[1]
    ed = edge_attr.shape[1]
    h = W1.shape[1]

    quant = _NW * _CHUNK
    e_pad = ((e + quant - 1) // quant) * quant
    pad = e_pad - e
    n_ch = e_pad // _NW // _CHUNK

    src = edge_index[0]
    dst = edge_index[1]
    fill = (jnp.arange(pad, dtype=jnp.int32) * 61) % n
    src_p = jnp.concatenate([src, fill]).reshape(_NW, n_ch, _CHUNK)
    dst_p = jnp.concatenate([dst, fill])

    tn = 1024
    split = ((n + _NC * tn - 1) // (_NC * tn)) * tn
    n_acc = ((split + _CHUNK + 15) // 16) * 16
    dump = split + (jnp.arange(e_pad, dtype=jnp.int32) % _CHUNK)
    idx_sc = []
    for c in range(_NC):
        local = dst_p - c * split
        valid = (local >= 0) & (dst_p < min(n, (c + 1) * split))
        idx_sc.append(jnp.where(valid, local, dump))
    idx_sc = jnp.stack(idx_sc).reshape(_NC, _NS, e_pad // _NS // _CHUNK,
                                       _CHUNK)

    ea_t = jnp.pad(edge_attr, ((0, pad), (0, 0))).T
    w2pt = W2.reshape(h, d, d).transpose(2, 1, 0).reshape(d * d, h)
    b2p = b2.reshape(d, d).T.reshape(d * d, 1)

    x_src = _make_gather(n, d, e_pad)(x, src_p)
    xs_t = x_src.T

    grid = e_pad // _TE
    msg_t = pl.pallas_call(
        functools.partial(_msg_body, d=d, e=e, te=_TE),
        grid=(grid,),
        in_specs=[
            pl.BlockSpec((ed, _TE), lambda i: (0, i)),
            pl.BlockSpec((d, _TE), lambda i: (0, i)),
            pl.BlockSpec((h, ed), lambda i: (0, 0)),
            pl.BlockSpec((h, 1), lambda i: (0, 0)),
            pl.BlockSpec((d * d, h), lambda i: (0, 0)),
            pl.BlockSpec((d * d, 1), lambda i: (0, 0)),
        ],
        out_specs=pl.BlockSpec((d, _TE), lambda i: (0, i)),
        out_shape=jax.ShapeDtypeStruct((d, e_pad), jnp.float32),
    )(ea_t, xs_t, W1.T, b1.reshape(h, 1), w2pt.astype(jnp.bfloat16), b2p)
    messages = msg_t.T

    zeros = jnp.zeros((n_acc // _NS, d), jnp.float32)
    agg = _make_scatter(n_acc, d, e_pad)(messages, idx_sc, zeros)

    hb = split // tn
    agg_t = jnp.transpose(agg, (0, 2, 1))
    out_t = pl.pallas_call(
        functools.partial(_gru_body, d=d),
        grid=((n + tn - 1) // tn,),
        in_specs=[
            pl.BlockSpec((d, tn), lambda i: (0, i)),
            pl.BlockSpec((1, d, tn), lambda i: (i // hb, 0, i % hb)),
            pl.BlockSpec((3 * d, d), lambda i: (0, 0)),
            pl.BlockSpec((3 * d, 1), lambda i: (0, 0)),
            pl.BlockSpec((3 * d, d), lambda i: (0, 0)),
            pl.BlockSpec((3 * d, 1), lambda i: (0, 0)),
        ],
        out_specs=pl.BlockSpec((d, tn), lambda i: (0, i)),
        out_shape=jax.ShapeDtypeStruct((d, n), jnp.float32),
    )(x.T, agg_t, W_ih.T, b_ih.reshape(3 * d, 1),
      W_hh.T, b_hh.reshape(3 * d, 1))
    return out_t.T


# speedup vs baseline: 1.0500x; 1.0331x over previous
import functools

import jax
import jax.numpy as jnp
from jax import lax
from jax.experimental import pallas as pl
from jax.experimental.pallas import tpu as pltpu
from jax.experimental.pallas import tpu_sc as plsc

_NC = 2
_NS = 16
_NW = _NC * _NS
_CHUNK = 128
_TE = 2048


def _msg_body(eat_ref, xst_ref, w1t_ref, b1_ref, w2pt_ref, b2p_ref, o_ref,
              *, d, e, te):
    h1 = jnp.dot(w1t_ref[...], eat_ref[...],
                 preferred_element_type=jnp.float32) + b1_ref[...]
    hid = 0.5 * h1 * (1.0 + lax.erf(h1 * 0.7071067811865476))
    q = jnp.dot(w2pt_ref[...], hid.astype(jnp.bfloat16),
                preferred_element_type=jnp.float32) + b2p_ref[...]
    q3 = q.reshape(d, d, te)
    xs3 = xst_ref[...].reshape(d, 1, te)
    msg = jnp.sum(q3 * xs3, axis=0)
    col = pl.program_id(0) * te + lax.broadcasted_iota(jnp.int32, (d, te), 1)
    o_ref[...] = jnp.where(col < e, msg, 0.0)


def _gru_body(xt_ref, aggt_ref, wiht_ref, bih_ref, whht_ref, bhh_ref,
              o_ref, *, d):
    xt = xt_ref[...]
    aggt = aggt_ref[0]
    gi = jnp.dot(wiht_ref[...], aggt,
                 preferred_element_type=jnp.float32) + bih_ref[...]
    gh = jnp.dot(whht_ref[...], xt,
                 preferred_element_type=jnp.float32) + bhh_ref[...]
    r = jax.nn.sigmoid(gi[:d] + gh[:d])
    z = jax.nn.sigmoid(gi[d:2 * d] + gh[d:2 * d])
    hn = gh[2 * d:]
    n = jnp.tanh(gi[2 * d:] + r * hn)
    o_ref[...] = (1.0 - z) * n + z * xt


def _make_gather(n, d, e_pad):
    per_w = e_pad // _NW
    n_ch = per_w // _CHUNK
    mesh = plsc.VectorSubcoreMesh(core_axis_name="c", subcore_axis_name="s")

    @functools.partial(
        pl.kernel, mesh=mesh,
        out_type=jax.ShapeDtypeStruct((e_pad, d), jnp.float32),
        compiler_params=pltpu.CompilerParams(use_tc_tiling_on_sc=False),
        scratch_types=[
            pltpu.VMEM((n_ch, _CHUNK), jnp.int32),
            pltpu.VMEM((per_w, d), jnp.float32),
            pltpu.SemaphoreType.DMA,
            pltpu.SemaphoreType.DMA,
        ],
    )
    def gather_k(x_hbm, idx_hbm, out_hbm, idx_v, rows_v, sem, sem_out):
        w = lax.axis_index("c") * _NS + lax.axis_index("s")
        pltpu.sync_copy(idx_hbm.at[w], idx_v)
        cps = [
            pltpu.async_copy(x_hbm.at[idx_v.at[ch]],
                             rows_v.at[pl.ds(ch * _CHUNK, _CHUNK)], sem)
            for ch in range(n_ch)
        ]
        outs = []
        grp = 5 * _CHUNK
        for g in range(n_ch // 5):
            for ch in range(5 * g, 5 * g + 5):
                cps[ch].wait()
            outs.append(pltpu.async_copy(
                rows_v.at[pl.ds(g * grp, grp)],
                out_hbm.at[pl.ds(w * per_w + g * grp, grp)], sem_out))
        for cp in outs:
            cp.wait()

    return gather_k


def _make_scatter(n_acc, d, e_pad):
    per_w = e_pad // _NS
    n_st = 10
    stage = per_w // n_st
    n_ch = stage // _CHUNK
    rows_per_tile = n_acc // _NS
    mesh = plsc.VectorSubcoreMesh(core_axis_name="c", subcore_axis_name="s")

    @functools.partial(
        pl.kernel, mesh=mesh,
        out_type=jax.ShapeDtypeStruct((_NC, n_acc, d), jnp.float32),
        compiler_params=pltpu.CompilerParams(use_tc_tiling_on_sc=False),
        scratch_types=[
            pltpu.VMEM((2, stage, d), jnp.float32),
            pltpu.VMEM((n_st * n_ch, _CHUNK), jnp.int32),
            pltpu.VMEM_SHARED((n_acc, d), jnp.float32),
            pltpu.SemaphoreType.DMA,
            pltpu.SemaphoreType.DMA,
        ],
    )
    def scatter_k(msg_hbm, idx_hbm, zeros_hbm, out_hbm, upd_v, idx_v, agg_sh,
                  sem, sem_in):
        c = lax.axis_index("c")
        s = lax.axis_index("s")
        pltpu.sync_copy(zeros_hbm,
                        agg_sh.at[pl.ds(s * rows_per_tile, rows_per_tile)])
        pltpu.sync_copy(idx_hbm.at[c, s], idx_v)
        plsc.subcore_barrier()
        loads = [pltpu.async_copy(msg_hbm.at[pl.ds(s * per_w, stage)],
                                  upd_v.at[0], sem_in)]
        for st in range(n_st):
            if st + 1 < n_st:
                loads.append(pltpu.async_copy(
                    msg_hbm.at[pl.ds(s * per_w + (st + 1) * stage, stage)],
                    upd_v.at[(st + 1) % 2], sem_in))
            loads[st].wait()
            cps = [
                pltpu.async_copy(upd_v.at[st % 2, pl.ds(ch * _CHUNK, _CHUNK)],
                                 agg_sh.at[idx_v.at[st * n_ch + ch]], sem,
                                 add=True)
                for ch in range(n_ch)
            ]
            for cp in cps:
                cp.wait()
        plsc.subcore_barrier()
        pltpu.sync_copy(
            agg_sh.at[pl.ds(s * rows_per_tile, rows_per_tile)],
            out_hbm.at[c, pl.ds(s * rows_per_tile, rows_per_tile)])

    return scatter_k


def kernel(x, edge_index, edge_attr, W1, b1, W2, b2, W_ih, b_ih, W_hh, b_hh):
    n, d = x.shape
    e = edge_index.shape[1]
    ed = edge_attr.shape[1]
    h = W1.shape[1]

    quant = _NW * _CHUNK
    e_pad = ((e + quant - 1) // quant) * quant
    pad = e_pad - e
    n_ch = e_pad // _NW // _CHUNK

    src = edge_index[0]
    dst = edge_index[1]
    fill = (jnp.arange(pad, dtype=jnp.int32) * 61) % n
    src_p = jnp.concatenate([src, fill]).reshape(_NW, n_ch, _CHUNK)
    dst_p = jnp.concatenate([dst, fill])

    tn = 2048
    split = ((n + _NC * tn - 1) // (_NC * tn)) * tn
    n_acc = ((split + _CHUNK + 15) // 16) * 16
    dump = split + (jnp.arange(e_pad, dtype=jnp.int32) % _CHUNK)
    idx_sc = []
    for c in range(_NC):
        local = dst_p - c * split
        valid = (local >= 0) & (dst_p < min(n, (c + 1) * split))
        idx_sc.append(jnp.where(valid, local, dump))
    idx_sc = jnp.stack(idx_sc).reshape(_NC, _NS, e_pad // _NS // _CHUNK,
                                       _CHUNK)

    ea_t = jnp.pad(edge_attr, ((0, pad), (0, 0))).T
    w2pt = W2.reshape(h, d, d).transpose(2, 1, 0).reshape(d * d, h)
    b2p = b2.reshape(d, d).T.reshape(d * d, 1)

    x_src = _make_gather(n, d, e_pad)(x, src_p)
    xs_t = x_src.T

    grid = e_pad // _TE
    msg_t = pl.pallas_call(
        functools.partial(_msg_body, d=d, e=e, te=_TE),
        grid=(grid,),
        in_specs=[
            pl.BlockSpec((ed, _TE), lambda i: (0, i)),
            pl.BlockSpec((d, _TE), lambda i: (0, i)),
            pl.BlockSpec((h, ed), lambda i: (0, 0)),
            pl.BlockSpec((h, 1), lambda i: (0, 0)),
            pl.BlockSpec((d * d, h), lambda i: (0, 0)),
            pl.BlockSpec((d * d, 1), lambda i: (0, 0)),
        ],
        out_specs=pl.BlockSpec((d, _TE), lambda i: (0, i)),
        out_shape=jax.ShapeDtypeStruct((d, e_pad), jnp.float32),
    )(ea_t, xs_t, W1.T, b1.reshape(h, 1), w2pt.astype(jnp.bfloat16), b2p)
    messages = msg_t.T

    zeros = jnp.zeros((n_acc // _NS, d), jnp.float32)
    agg = _make_scatter(n_acc, d, e_pad)(messages, idx_sc, zeros)

    hb = split // tn
    agg_t = jnp.transpose(agg, (0, 2, 1))
    out_t = pl.pallas_call(
        functools.partial(_gru_body, d=d),
        grid=((n + tn - 1) // tn,),
        in_specs=[
            pl.BlockSpec((d, tn), lambda i: (0, i)),
            pl.BlockSpec((1, d, tn), lambda i: (i // hb, 0, i % hb)),
            pl.BlockSpec((3 * d, d), lambda i: (0, 0)),
            pl.BlockSpec((3 * d, 1), lambda i: (0, 0)),
            pl.BlockSpec((3 * d, d), lambda i: (0, 0)),
            pl.BlockSpec((3 * d, 1), lambda i: (0, 0)),
        ],
        out_specs=pl.BlockSpec((d, tn), lambda i: (0, i)),
        out_shape=jax.ShapeDtypeStruct((d, n), jnp.float32),
    )(x.T, agg_t, W_ih.T, b_ih.reshape(3 * d, 1),
      W_hh.T, b_hh.reshape(3 * d, 1))
    return out_t.T


# speedup vs baseline: 1.0601x; 1.0096x over previous
import functools

import jax
import jax.numpy as jnp
from jax import lax
from jax.experimental import pallas as pl
from jax.experimental.pallas import tpu as pltpu
from jax.experimental.pallas import tpu_sc as plsc

_NC = 2
_NS = 16
_NW = _NC * _NS
_CHUNK = 128
_TE = 2048


def _msg_body(eat_ref, xst_ref, w1t_ref, b1_ref, w2pt_ref, b2p_ref, o_ref,
              *, d, e, te):
    h1 = jnp.dot(w1t_ref[...], eat_ref[...],
                 preferred_element_type=jnp.float32) + b1_ref[...]
    hid = 0.5 * h1 * (1.0 + lax.erf(h1 * 0.7071067811865476))
    q = jnp.dot(w2pt_ref[...], hid.astype(jnp.bfloat16),
                preferred_element_type=jnp.float32) + b2p_ref[...]
    q3 = q.reshape(d, d, te)
    xs3 = xst_ref[...].reshape(d, 1, te)
    msg = jnp.sum(q3 * xs3, axis=0)
    col = pl.program_id(0) * te + lax.broadcasted_iota(jnp.int32, (d, te), 1)
    o_ref[...] = jnp.where(col < e, msg, 0.0)


def _gru_body(xt_ref, aggt_ref, wiht_ref, bih_ref, whht_ref, bhh_ref,
              o_ref, *, d):
    xt = xt_ref[...]
    aggt = aggt_ref[0]
    gi = jnp.dot(wiht_ref[...], aggt,
                 preferred_element_type=jnp.float32) + bih_ref[...]
    gh = jnp.dot(whht_ref[...], xt,
                 preferred_element_type=jnp.float32) + bhh_ref[...]
    r = jax.nn.sigmoid(gi[:d] + gh[:d])
    z = jax.nn.sigmoid(gi[d:2 * d] + gh[d:2 * d])
    hn = gh[2 * d:]
    n = jnp.tanh(gi[2 * d:] + r * hn)
    o_ref[...] = (1.0 - z) * n + z * xt


def _make_gather(n, d, e_pad):
    per_w = e_pad // _NW
    n_ch = per_w // _CHUNK
    mesh = plsc.VectorSubcoreMesh(core_axis_name="c", subcore_axis_name="s")

    @functools.partial(
        pl.kernel, mesh=mesh,
        out_type=jax.ShapeDtypeStruct((e_pad, d), jnp.float32),
        compiler_params=pltpu.CompilerParams(use_tc_tiling_on_sc=False),
        scratch_types=[
            pltpu.VMEM((n_ch, _CHUNK), jnp.int32),
            pltpu.VMEM((per_w, d), jnp.float32),
            pltpu.SemaphoreType.DMA,
            pltpu.SemaphoreType.DMA,
        ],
    )
    def gather_k(x_hbm, idx_hbm, out_hbm, idx_v, rows_v, sem, sem_out):
        w = lax.axis_index("c") * _NS + lax.axis_index("s")
        pltpu.sync_copy(idx_hbm.at[w], idx_v)
        cps = [
            pltpu.async_copy(x_hbm.at[idx_v.at[ch]],
                             rows_v.at[pl.ds(ch * _CHUNK, _CHUNK)], sem)
            for ch in range(n_ch)
        ]
        outs = []
        grp = 5 * _CHUNK
        for g in range(n_ch // 5):
            for ch in range(5 * g, 5 * g + 5):
                cps[ch].wait()
            outs.append(pltpu.async_copy(
                rows_v.at[pl.ds(g * grp, grp)],
                out_hbm.at[pl.ds(w * per_w + g * grp, grp)], sem_out))
        for cp in outs:
            cp.wait()

    return gather_k


def _make_scatter(n_acc, d, e_pad):
    per_w = e_pad // _NS
    n_st = 10
    stage = per_w // n_st
    n_ch = stage // _CHUNK
    rows_per_tile = n_acc // _NS
    mesh = plsc.VectorSubcoreMesh(core_axis_name="c", subcore_axis_name="s")

    @functools.partial(
        pl.kernel, mesh=mesh,
        out_type=jax.ShapeDtypeStruct((_NC, n_acc, d), jnp.float32),
        compiler_params=pltpu.CompilerParams(use_tc_tiling_on_sc=False),
        scratch_types=[
            pltpu.VMEM((2, stage, d), jnp.float32),
            pltpu.VMEM((n_st * n_ch, _CHUNK), jnp.int32),
            pltpu.VMEM_SHARED((n_acc, d), jnp.float32),
            pltpu.SemaphoreType.DMA,
            pltpu.SemaphoreType.DMA,
        ],
    )
    def scatter_k(msg_hbm, idx_hbm, zeros_hbm, out_hbm, upd_v, idx_v, agg_sh,
                  sem, sem_in):
        c = lax.axis_index("c")
        s = lax.axis_index("s")
        pltpu.sync_copy(zeros_hbm,
                        agg_sh.at[pl.ds(s * rows_per_tile, rows_per_tile)])
        pltpu.sync_copy(idx_hbm.at[c, s], idx_v)
        plsc.subcore_barrier()
        loads = [pltpu.async_copy(msg_hbm.at[pl.ds(s * per_w, stage)],
                                  upd_v.at[0], sem_in)]
        for st in range(n_st):
            if st + 1 < n_st:
                loads.append(pltpu.async_copy(
                    msg_hbm.at[pl.ds(s * per_w + (st + 1) * stage, stage)],
                    upd_v.at[(st + 1) % 2], sem_in))
            loads[st].wait()
            cps = [
                pltpu.async_copy(upd_v.at[st % 2, pl.ds(ch * _CHUNK, _CHUNK)],
                                 agg_sh.at[idx_v.at[st * n_ch + ch]], sem,
                                 add=True)
                for ch in range(n_ch)
            ]
            for cp in cps:
                cp.wait()
        plsc.subcore_barrier()
        pltpu.sync_copy(
            agg_sh.at[pl.ds(s * rows_per_tile, rows_per_tile)],
            out_hbm.at[c, pl.ds(s * rows_per_tile, rows_per_tile)])

    return scatter_k


def kernel(x, edge_index, edge_attr, W1, b1, W2, b2, W_ih, b_ih, W_hh, b_hh):
    n, d = x.shape
    e = edge_index.shape[1]
    ed = edge_attr.shape[1]
    h = W1.shape[1]

    quant = _NW * _CHUNK
    e_pad = ((e + quant - 1) // quant) * quant
    pad = e_pad - e
    n_ch = e_pad // _NW // _CHUNK

    src = edge_index[0]
    dst = edge_index[1]
    fill = (jnp.arange(pad, dtype=jnp.int32) * 61) % n
    src_p = jnp.concatenate([src, fill]).reshape(_NW, n_ch, _CHUNK)
    dst_p = jnp.concatenate([dst, fill])

    tn = 4096
    split = ((n + _NC * tn - 1) // (_NC * tn)) * tn
    n_acc = ((split + _CHUNK + 15) // 16) * 16
    dump = split + (jnp.arange(e_pad, dtype=jnp.int32) % _CHUNK)
    idx_sc = []
    for c in range(_NC):
        local = dst_p - c * split
        valid = (local >= 0) & (dst_p < min(n, (c + 1) * split))
        idx_sc.append(jnp.where(valid, local, dump))
    idx_sc = jnp.stack(idx_sc).reshape(_NC, _NS, e_pad // _NS // _CHUNK,
                                       _CHUNK)

    ea_t = jnp.pad(edge_attr, ((0, pad), (0, 0))).T
    w2pt = W2.reshape(h, d, d).transpose(2, 1, 0).reshape(d * d, h)
    b2p = b2.reshape(d, d).T.reshape(d * d, 1)

    x_src = _make_gather(n, d, e_pad)(x, src_p)
    xs_t = x_src.T

    grid = e_pad // _TE
    msg_t = pl.pallas_call(
        functools.partial(_msg_body, d=d, e=e, te=_TE),
        grid=(grid,),
        in_specs=[
            pl.BlockSpec((ed, _TE), lambda i: (0, i)),
            pl.BlockSpec((d, _TE), lambda i: (0, i)),
            pl.BlockSpec((h, ed), lambda i: (0, 0)),
            pl.BlockSpec((h, 1), lambda i: (0, 0)),
            pl.BlockSpec((d * d, h), lambda i: (0, 0)),
            pl.BlockSpec((d * d, 1), lambda i: (0, 0)),
        ],
        out_specs=pl.BlockSpec((d, _TE), lambda i: (0, i)),
        out_shape=jax.ShapeDtypeStruct((d, e_pad), jnp.float32),
    )(ea_t, xs_t, W1.T, b1.reshape(h, 1), w2pt.astype(jnp.bfloat16), b2p)
    messages = msg_t.T

    zeros = jnp.zeros((n_acc // _NS, d), jnp.float32)
    agg = _make_scatter(n_acc, d, e_pad)(messages, idx_sc, zeros)

    hb = split // tn
    agg_t = jnp.transpose(agg, (0, 2, 1))
    out_t = pl.pallas_call(
        functools.partial(_gru_body, d=d),
        grid=((n + tn - 1) // tn,),
        in_specs=[
            pl.BlockSpec((d, tn), lambda i: (0, i)),
            pl.BlockSpec((1, d, tn), lambda i: (i // hb, 0, i % hb)),
            pl.BlockSpec((3 * d, d), lambda i: (0, 0)),
            pl.BlockSpec((3 * d, 1), lambda i: (0, 0)),
            pl.BlockSpec((3 * d, d), lambda i: (0, 0)),
            pl.BlockSpec((3 * d, 1), lambda i: (0, 0)),
        ],
        out_specs=pl.BlockSpec((d, tn), lambda i: (0, i)),
        out_shape=jax.ShapeDtypeStruct((d, n), jnp.float32),
    )(x.T, agg_t, W_ih.T, b_ih.reshape(3 * d, 1),
      W_hh.T, b_hh.reshape(3 * d, 1))
    return out_t.T
